# Initial kernel scaffold; baseline (speedup 1.0000x reference)
import jax
import jax.numpy as jnp
from jax.experimental import pallas as pl


def kernel(x_cat, weight):
    raise NotImplementedError("write your pallas kernel here")


# speedup vs baseline: 3.2413x; 3.2413x over previous
import functools

import jax
import jax.numpy as jnp
from jax import lax
from jax.experimental import pallas as pl
from jax.experimental.pallas import tpu as pltpu
from jax.experimental.pallas import tpu_sc as plsc

NUM_FEATURES = 26
CAT_SIZE = 1000
D_EMBED = 128
BATCH = 16384
TOTAL = BATCH * NUM_FEATURES

NC = 2
NS = 16
NW = NC * NS
PER_W = TOTAL // NW
K_ROWS = 8
CHUNK = K_ROWS * 128
HALF = CHUNK // 2
N_CHUNKS = PER_W // CHUNK


def _sc_embedding_gather(x2d, weight):
    mesh = plsc.VectorSubcoreMesh(core_axis_name="c", subcore_axis_name="s")

    @functools.partial(
        pl.kernel,
        mesh=mesh,
        out_type=jax.ShapeDtypeStruct((TOTAL, D_EMBED), jnp.float32),
        scratch_types=[
            pltpu.VMEM((K_ROWS, 128), jnp.int32),
            pltpu.VMEM((HALF, D_EMBED), jnp.float32),
            pltpu.SemaphoreType.DMA,
        ],
    )
    def body(x_hbm, w_hbm, out_hbm, idx_v, rows_v, sem):
        wid = lax.axis_index("s") * NC + lax.axis_index("c")
        base = wid * PER_W

        def chunk_body(c, carry):
            off = pl.multiple_of(base + c * CHUNK, CHUNK)
            row0 = pl.multiple_of(off // 128, K_ROWS)
            pltpu.sync_copy(x_hbm.at[pl.ds(row0, K_ROWS)], idx_v)
            for j in range(K_ROWS):
                for p in range(8):
                    pos = off + j * 128 + p * 16 + lax.iota(jnp.int32, 16)
                    feat = lax.rem(pos, NUM_FEATURES)
                    sl = pl.ds(p * 16, 16)
                    idx_v[j, sl] = idx_v[j, sl] + feat * CAT_SIZE
            for half in range(2):
                copies = [
                    pltpu.async_copy(
                        w_hbm.at[idx_v.at[half * 4 + j]],
                        rows_v.at[pl.ds(j * 128, 128)],
                        sem,
                    )
                    for j in range(4)
                ]
                for cp in copies:
                    cp.wait()
                out0 = pl.multiple_of(off + half * HALF, HALF)
                pltpu.sync_copy(rows_v, out_hbm.at[pl.ds(out0, HALF)])
            return carry

        lax.fori_loop(0, N_CHUNKS, chunk_body, 0)

    return body(x2d, weight)


def kernel(x_cat, weight):
    x2d = x_cat.reshape(TOTAL // 128, 128)
    out = _sc_embedding_gather(x2d, weight)
    return out.reshape(BATCH, NUM_FEATURES, D_EMBED)


# speedup vs baseline: 3.3463x; 1.0324x over previous
import functools

import jax
import jax.numpy as jnp
from jax import lax
from jax.experimental import pallas as pl
from jax.experimental.pallas import tpu as pltpu
from jax.experimental.pallas import tpu_sc as plsc

NUM_FEATURES = 26
CAT_SIZE = 1000
D_EMBED = 128
BATCH = 16384
TOTAL = BATCH * NUM_FEATURES

NC = 2
NS = 16
NW = NC * NS
PER_W = TOTAL // NW
K_ROWS = 8
CHUNK = K_ROWS * 128
STEP = 256
N_STEPS = CHUNK // STEP
N_CHUNKS = PER_W // CHUNK


def _sc_embedding_gather(x2d, weight):
    mesh = plsc.VectorSubcoreMesh(core_axis_name="c", subcore_axis_name="s")

    @functools.partial(
        pl.kernel,
        mesh=mesh,
        out_type=jax.ShapeDtypeStruct((TOTAL, D_EMBED), jnp.float32),
        scratch_types=[
            pltpu.VMEM((K_ROWS, 128), jnp.int32),
            pltpu.VMEM((STEP, D_EMBED), jnp.float32),
            pltpu.VMEM((STEP, D_EMBED), jnp.float32),
            pltpu.SemaphoreType.DMA,
            pltpu.SemaphoreType.DMA,
            pltpu.SemaphoreType.DMA,
            pltpu.SemaphoreType.DMA,
        ],
    )
    def body(x_hbm, w_hbm, out_hbm, idx_v, bufa, bufb, sga, sgb, swa, swb):
        wid = lax.axis_index("s") * NC + lax.axis_index("c")
        base = wid * PER_W
        bufs = (bufa, bufb)
        sgs = (sga, sgb)
        sws = (swa, swb)

        def fire_gathers(q, buf, sg):
            for j in range(2):
                pltpu.async_copy(
                    w_hbm.at[idx_v.at[2 * q + j]],
                    buf.at[pl.ds(j * 128, 128)],
                    sg,
                )

        def wait_gathers(q, buf, sg):
            for j in range(2):
                pltpu.make_async_copy(
                    w_hbm.at[idx_v.at[2 * q + j]],
                    buf.at[pl.ds(j * 128, 128)],
                    sg,
                ).wait()

        def chunk_body(c, carry):
            off = pl.multiple_of(base + c * CHUNK, CHUNK)
            row0 = pl.multiple_of(off // 128, K_ROWS)
            pltpu.sync_copy(x_hbm.at[pl.ds(row0, K_ROWS)], idx_v)
            for j in range(K_ROWS):
                for p in range(8):
                    pos = off + j * 128 + p * 16 + lax.iota(jnp.int32, 16)
                    feat = lax.rem(pos, NUM_FEATURES)
                    sl = pl.ds(p * 16, 16)
                    idx_v[j, sl] = idx_v[j, sl] + feat * CAT_SIZE
            for q in range(N_STEPS):
                b = q % 2
                step_off = pl.multiple_of(off + q * STEP, STEP)
                wb_wait = lambda: pltpu.make_async_copy(
                    bufs[b], out_hbm.at[pl.ds(step_off, STEP)], sws[b]
                ).wait()
                if q >= 2:
                    wb_wait()
                else:
                    pl.when(c > 0)(wb_wait)
                fire_gathers(q, bufs[b], sgs[b])
                if q >= 1:
                    bp = (q - 1) % 2
                    prev_off = pl.multiple_of(off + (q - 1) * STEP, STEP)
                    wait_gathers(q - 1, bufs[bp], sgs[bp])
                    pltpu.async_copy(
                        bufs[bp], out_hbm.at[pl.ds(prev_off, STEP)], sws[bp]
                    )
            qL = N_STEPS - 1
            bL = qL % 2
            last_off = pl.multiple_of(off + qL * STEP, STEP)
            wait_gathers(qL, bufs[bL], sgs[bL])
            pltpu.async_copy(bufs[bL], out_hbm.at[pl.ds(last_off, STEP)], sws[bL])
            return carry

        lax.fori_loop(0, N_CHUNKS, chunk_body, 0)

        tail = pl.multiple_of(base + (N_CHUNKS - 1) * CHUNK, CHUNK)
        offa = pl.multiple_of(tail + 2 * STEP, STEP)
        offb = pl.multiple_of(tail + 3 * STEP, STEP)
        pltpu.make_async_copy(bufa, out_hbm.at[pl.ds(offa, STEP)], swa).wait()
        pltpu.make_async_copy(bufb, out_hbm.at[pl.ds(offb, STEP)], swb).wait()

    return body(x2d, weight)


def kernel(x_cat, weight):
    x2d = x_cat.reshape(TOTAL // 128, 128)
    out = _sc_embedding_gather(x2d, weight)
    return out.reshape(BATCH, NUM_FEATURES, D_EMBED)
